# baseline (speedup 1.0000x reference)
import numpy as np
import jax
import jax.numpy as jnp
from jax.experimental import pallas as pl
from jax.experimental.pallas import tpu as pltpu

_B = 8
_C = 8
_O = 128
_K = 1024
_L = 131072
_POS_PER_STEP = 16
_STEPS = _O // _POS_PER_STEP
_NF = _K // 2 + 1


def _dft_constants():
    n = _K
    t = np.arange(n, dtype=np.float64)
    win = 0.5 * (1.0 - np.cos(2.0 * np.pi * t / n))
    f = np.arange(_NF, dtype=np.float64)
    ang = 2.0 * np.pi * np.outer(t, f) / n
    a = win[:, None] * np.cos(ang)
    b = win[:, None] * np.sin(ang)
    c = np.full(_NF, 2.0, dtype=np.float64)
    c[0] = 1.0
    c[-1] = 1.0
    j = np.arange(n, dtype=np.float64)
    d = ((c * ((-1.0) ** f))[:, None] * np.cos(2.0 * np.pi * np.outer(f, j) / n)
         / n)
    return a, b, d


_A_CONST, _B_CONST, _D_CONST = _dft_constants()


_WIDE = _K + 128


def _dma_copies(pos_ref, x_hbm, scratch, sem, step, slot):
    copies = []
    for j in range(_POS_PER_STEP):
        pos = pos_ref[step * _POS_PER_STEP + j]
        al = (pos // 128) * 128
        for bh in range(2):
            src = x_hbm.at[pl.ds(4 * (1 - bh), 4), :, pl.ds(al, _WIDE)]
            dst = scratch.at[slot, pl.ds(4 * bh, 4), j, :, :]
            copies.append(pltpu.make_async_copy(src, dst, sem.at[slot]))
    return copies


def _body(pos_ref, x_hbm, a_ref, b_ref, d_ref, out_ref, scratch, sem):
    i = pl.program_id(0)
    slot = jax.lax.rem(i, 2)

    @pl.when(i == 0)
    def _():
        for cp in _dma_copies(pos_ref, x_hbm, scratch, sem, i, slot):
            cp.start()

    @pl.when(i + 1 < _STEPS)
    def _():
        for cp in _dma_copies(pos_ref, x_hbm, scratch, sem, i + 1, 1 - slot):
            cp.start()

    for cp in _dma_copies(pos_ref, x_hbm, scratch, sem, i, slot):
        cp.wait()

    rows = []
    lane = jax.lax.broadcasted_iota(jnp.int32, (_B * _C, _WIDE), 1)
    for j in range(_POS_PER_STEP):
        pos = pos_ref[i * _POS_PER_STEP + j]
        shift = pos - (pos // 128) * 128
        wide = scratch[slot, :, j, :, :].reshape(_B * _C, _WIDE)
        mask = (lane >= shift) & (lane < shift + _K)
        mj = jnp.max(jnp.where(mask, jnp.abs(wide), 0.0), axis=-1,
                     keepdims=True)
        mj = jnp.maximum(mj, jnp.finfo(jnp.float32).eps)
        norm = wide * (1.0 / mj)
        rot = pltpu.roll(norm.astype(jnp.bfloat16), _WIDE - shift, axis=1)
        rows.append(rot[:, :_K].reshape(_B, 1, _C, _K))
    w = jnp.concatenate(rows, axis=1).reshape(_B * _POS_PER_STEP * _C, _K)

    re = jnp.dot(w, a_ref[...], preferred_element_type=jnp.float32)
    im = jnp.dot(w, b_ref[...], preferred_element_type=jnp.float32)
    p = (re * re + im * im).astype(jnp.bfloat16)
    out = jnp.dot(p, d_ref[...], preferred_element_type=jnp.float32)
    out4 = out.reshape(_B, _POS_PER_STEP, _C, _K)
    out_ref[:, :, 0:4, :] = out4[:, :, 4:8, :]
    out_ref[:, :, 4:8, :] = out4[:, :, 0:4, :]


def kernel(X, position_concentration):
    del position_concentration
    end = _L - _K - 1 - _K
    positions = jnp.linspace(0.0, float(end), _O).astype(jnp.int32)

    grid_spec = pltpu.PrefetchScalarGridSpec(
        num_scalar_prefetch=1,
        grid=(_STEPS,),
        in_specs=[
            pl.BlockSpec(memory_space=pltpu.MemorySpace.HBM),
            pl.BlockSpec((_K, _NF), lambda i, pos: (0, 0)),
            pl.BlockSpec((_K, _NF), lambda i, pos: (0, 0)),
            pl.BlockSpec((_NF, _K), lambda i, pos: (0, 0)),
        ],
        out_specs=pl.BlockSpec(
            (_B, _POS_PER_STEP, _C, _K),
            lambda i, pos: (0, (i + _STEPS // 2) % _STEPS, 0, 0)),
        scratch_shapes=[
            pltpu.VMEM((2, _B, _POS_PER_STEP, _C, _WIDE), jnp.float32),
            pltpu.SemaphoreType.DMA((2,)),
        ],
    )
    out = pl.pallas_call(
        _body,
        grid_spec=grid_spec,
        out_shape=jax.ShapeDtypeStruct((_B, _O, _C, _K), jnp.float32),
    )(positions, X, jnp.asarray(_A_CONST, dtype=jnp.bfloat16),
      jnp.asarray(_B_CONST, dtype=jnp.bfloat16),
      jnp.asarray(_D_CONST, dtype=jnp.bfloat16))
    return out


# speedup vs baseline: 1.0649x; 1.0649x over previous
import numpy as np
import jax
import jax.numpy as jnp
from jax.experimental import pallas as pl
from jax.experimental.pallas import tpu as pltpu

_B = 8
_C = 8
_O = 128
_K = 1024
_L = 131072
_POS_PER_STEP = 8
_STEPS = _O // _POS_PER_STEP
_NF = _K // 2 + 1


def _dft_constants():
    n = _K
    t = np.arange(n, dtype=np.float64)
    win = 0.5 * (1.0 - np.cos(2.0 * np.pi * t / n))
    f = np.arange(_NF, dtype=np.float64)
    ang = 2.0 * np.pi * np.outer(t, f) / n
    a = win[:, None] * np.cos(ang)
    b = win[:, None] * np.sin(ang)
    c = np.full(_NF, 2.0, dtype=np.float64)
    c[0] = 1.0
    c[-1] = 1.0
    j = np.arange(n, dtype=np.float64)
    d = ((c * ((-1.0) ** f))[:, None] * np.cos(2.0 * np.pi * np.outer(f, j) / n)
         / n)
    return a, b, d


_A_CONST, _B_CONST, _D_CONST = _dft_constants()


_WIDE = _K + 128


def _dma_copies(pos_ref, x_hbm, scratch, sem, step, slot):
    copies = []
    for j in range(_POS_PER_STEP):
        pos = pos_ref[step * _POS_PER_STEP + j]
        al = (pos // 128) * 128
        for bh in range(2):
            src = x_hbm.at[pl.ds(4 * (1 - bh), 4), :, pl.ds(al, _WIDE)]
            dst = scratch.at[slot, pl.ds(4 * bh, 4), j, :, :]
            copies.append(pltpu.make_async_copy(src, dst, sem.at[slot]))
    return copies


def _body(pos_ref, x_hbm, a_ref, b_ref, d_ref, out_ref, scratch, sem):
    i = pl.program_id(0)
    slot = jax.lax.rem(i, 2)

    @pl.when(i == 0)
    def _():
        for cp in _dma_copies(pos_ref, x_hbm, scratch, sem, i, slot):
            cp.start()

    @pl.when(i + 1 < _STEPS)
    def _():
        for cp in _dma_copies(pos_ref, x_hbm, scratch, sem, i + 1, 1 - slot):
            cp.start()

    wait_cp = pltpu.make_async_copy(
        x_hbm.at[pl.ds(0, 4), :, pl.ds(0, _WIDE)],
        scratch.at[slot, pl.ds(0, 4), 0, :, :], sem.at[slot])
    for _ in range(2 * _POS_PER_STEP):
        wait_cp.wait()

    lane = jax.lax.broadcasted_iota(jnp.int32, (_B * _C, _WIDE), 1)
    half_j = _POS_PER_STEP // 2
    for h in range(2):
        rows = []
        for j in range(h * half_j, (h + 1) * half_j):
            pos = pos_ref[i * _POS_PER_STEP + j]
            shift = pos - (pos // 128) * 128
            wide = scratch[slot, :, j, :, :].reshape(_B * _C, _WIDE)
            mask = (lane >= shift) & (lane < shift + _K)
            mj = jnp.max(jnp.where(mask, jnp.abs(wide), 0.0), axis=-1,
                         keepdims=True)
            mj = jnp.maximum(mj, jnp.finfo(jnp.float32).eps)
            norm = wide * (1.0 / mj)
            rot = pltpu.roll(norm.astype(jnp.bfloat16), _WIDE - shift, axis=1)
            rows.append(rot[:, :_K].reshape(_B, 1, _C, _K))
        w = jnp.concatenate(rows, axis=1).reshape(_B * half_j * _C, _K)

        re = jnp.dot(w, a_ref[...], preferred_element_type=jnp.float32)
        im = jnp.dot(w, b_ref[...], preferred_element_type=jnp.float32)
        p = (re * re + im * im).astype(jnp.bfloat16)
        out = jnp.dot(p, d_ref[...], preferred_element_type=jnp.float32)
        out4 = out.reshape(_B, half_j, _C, _K)
        out_ref[:, h * half_j:(h + 1) * half_j, 0:4, :] = out4[:, :, 4:8, :]
        out_ref[:, h * half_j:(h + 1) * half_j, 4:8, :] = out4[:, :, 0:4, :]


def kernel(X, position_concentration):
    del position_concentration
    end = _L - _K - 1 - _K
    positions = jnp.linspace(0.0, float(end), _O).astype(jnp.int32)

    grid_spec = pltpu.PrefetchScalarGridSpec(
        num_scalar_prefetch=1,
        grid=(_STEPS,),
        in_specs=[
            pl.BlockSpec(memory_space=pltpu.MemorySpace.HBM),
            pl.BlockSpec((_K, _NF), lambda i, pos: (0, 0)),
            pl.BlockSpec((_K, _NF), lambda i, pos: (0, 0)),
            pl.BlockSpec((_NF, _K), lambda i, pos: (0, 0)),
        ],
        out_specs=pl.BlockSpec(
            (_B, _POS_PER_STEP, _C, _K),
            lambda i, pos: (0, (i + _STEPS // 2) % _STEPS, 0, 0)),
        scratch_shapes=[
            pltpu.VMEM((2, _B, _POS_PER_STEP, _C, _WIDE), jnp.float32),
            pltpu.SemaphoreType.DMA((2,)),
        ],
    )
    out = pl.pallas_call(
        _body,
        grid_spec=grid_spec,
        out_shape=jax.ShapeDtypeStruct((_B, _O, _C, _K), jnp.float32),
    )(positions, X, jnp.asarray(_A_CONST, dtype=jnp.bfloat16),
      jnp.asarray(_B_CONST, dtype=jnp.bfloat16),
      jnp.asarray(_D_CONST, dtype=jnp.bfloat16))
    return out
